# baseline (speedup 1.0000x reference)
import functools

import jax
import jax.numpy as jnp
from jax.experimental import pallas as pl

_NUM_EMBEDDINGS = 512
_DIM = 32
_COMMITMENT_COST = 0.25
_BLOCK = 2000


def _vq_block_kernel(h_ref, cb_ref, q_ref, idx_ref, loss_ref):
    h = h_ref[...]
    cb = cb_ref[...]
    hh = jnp.sum(h * h, axis=1, keepdims=True)
    cc = jnp.sum(cb * cb, axis=1)
    cross = jax.lax.dot_general(
        h, cb, (((1,), (1,)), ((), ())),
        preferred_element_type=jnp.float32)
    d = (hh + cc[None, :]) - 2.0 * cross
    dmin = jnp.min(d, axis=1, keepdims=True)
    iota = jax.lax.broadcasted_iota(jnp.int32, d.shape, 1)
    idx = jnp.min(jnp.where(d <= dmin, iota, _NUM_EMBEDDINGS), axis=1)
    onehot = (iota == idx[:, None]).astype(jnp.float32)
    q = jax.lax.dot_general(
        onehot, cb, (((1,), (0,)), ((), ())),
        preferred_element_type=jnp.float32)
    q_ref[...] = q
    idx_ref[...] = idx[:, None]
    err = jnp.sum((q - h) ** 2, axis=1, keepdims=True)
    part = jnp.sum(err, axis=0, keepdims=True)
    prev = jnp.where(pl.program_id(0) == 0, 0.0, loss_ref[...])
    loss_ref[...] = prev + part


@functools.partial(jax.jit, static_argnames=())
def kernel(h_v_k, codebook):
    n, d = h_v_k.shape
    e = codebook.shape[0]
    grid = n // _BLOCK
    q, idx, loss = pl.pallas_call(
        _vq_block_kernel,
        grid=(grid,),
        in_specs=[
            pl.BlockSpec((_BLOCK, d), lambda i: (i, 0)),
            pl.BlockSpec((e, d), lambda i: (0, 0)),
        ],
        out_specs=[
            pl.BlockSpec((_BLOCK, d), lambda i: (i, 0)),
            pl.BlockSpec((_BLOCK, 1), lambda i: (i, 0)),
            pl.BlockSpec((1, 1), lambda i: (0, 0)),
        ],
        out_shape=[
            jax.ShapeDtypeStruct((n, d), jnp.float32),
            jax.ShapeDtypeStruct((n, 1), jnp.int32),
            jax.ShapeDtypeStruct((1, 1), jnp.float32),
        ],
    )(h_v_k, codebook)
    total_loss = loss[0, 0] * ((1.0 + _COMMITMENT_COST) / (n * d))
    return (q, idx.reshape(n), total_loss)


# speedup vs baseline: 1.1432x; 1.1432x over previous
import functools

import jax
import jax.numpy as jnp
from jax.experimental import pallas as pl

_NUM_EMBEDDINGS = 512
_DIM = 32
_COMMITMENT_COST = 0.25
_BLOCK = 2000


def _vq_block_kernel(h_ref, cb_ref, cc_ref, q_ref, idx_ref, loss_ref):
    h = h_ref[...]
    cb = cb_ref[...]
    hh = jnp.sum(h * h, axis=1, keepdims=True)
    cc = cc_ref[0, :]
    cross2 = jax.lax.dot_general(
        h * (-2.0), cb, (((1,), (1,)), ((), ())),
        preferred_element_type=jnp.float32)
    d = (hh + cc[None, :]) + cross2
    dmin = jnp.min(d, axis=1, keepdims=True)
    iota_f = jax.lax.broadcasted_iota(jnp.int32, d.shape, 1).astype(jnp.float32)
    idx_f = jnp.min(jnp.where(d <= dmin, iota_f, float(_NUM_EMBEDDINGS)),
                    axis=1, keepdims=True)
    idx = idx_f.astype(jnp.int32)[:, 0]
    onehot = (iota_f == idx_f).astype(jnp.float32)
    q = jax.lax.dot_general(
        onehot, cb, (((1,), (0,)), ((), ())),
        preferred_element_type=jnp.float32)
    q_ref[...] = q
    idx_ref[...] = idx[:, None]
    part = jnp.sum(dmin, axis=0, keepdims=True)
    prev = jnp.where(pl.program_id(0) == 0, 0.0, loss_ref[...])
    loss_ref[...] = prev + part


@functools.partial(jax.jit, static_argnames=())
def kernel(h_v_k, codebook):
    n, d = h_v_k.shape
    e = codebook.shape[0]
    cc = jnp.sum(codebook * codebook, axis=1)[None, :]
    grid = n // _BLOCK
    q, idx, loss = pl.pallas_call(
        _vq_block_kernel,
        grid=(grid,),
        in_specs=[
            pl.BlockSpec((_BLOCK, d), lambda i: (i, 0)),
            pl.BlockSpec((e, d), lambda i: (0, 0)),
            pl.BlockSpec((1, e), lambda i: (0, 0)),
        ],
        out_specs=[
            pl.BlockSpec((_BLOCK, d), lambda i: (i, 0)),
            pl.BlockSpec((_BLOCK, 1), lambda i: (i, 0)),
            pl.BlockSpec((1, 1), lambda i: (0, 0)),
        ],
        out_shape=[
            jax.ShapeDtypeStruct((n, d), jnp.float32),
            jax.ShapeDtypeStruct((n, 1), jnp.int32),
            jax.ShapeDtypeStruct((1, 1), jnp.float32),
        ],
    )(h_v_k, codebook, cc)
    total_loss = loss[0, 0] * ((1.0 + _COMMITMENT_COST) / (n * d))
    return (q, idx.reshape(n), total_loss)


# speedup vs baseline: 1.2064x; 1.0553x over previous
import functools

import jax
import jax.numpy as jnp
from jax.experimental import pallas as pl

_NUM_EMBEDDINGS = 512
_DIM = 32
_COMMITMENT_COST = 0.25
_BLOCK = 4000


def _vq_block_kernel(h_ref, cb_ref, cc_ref, q_ref, idx_ref, loss_ref):
    h = h_ref[...]
    cb = cb_ref[...]
    hh = jnp.sum(h * h, axis=1, keepdims=True)
    cc = cc_ref[0, :]
    cross2 = jax.lax.dot_general(
        h * (-2.0), cb, (((1,), (1,)), ((), ())),
        preferred_element_type=jnp.float32)
    d = (hh + cc[None, :]) + cross2
    dmin = jnp.min(d, axis=1, keepdims=True)
    iota_f = jax.lax.broadcasted_iota(jnp.int32, d.shape, 1).astype(jnp.float32)
    idx_f = jnp.min(jnp.where(d <= dmin, iota_f, float(_NUM_EMBEDDINGS)),
                    axis=1, keepdims=True)
    idx = idx_f.astype(jnp.int32)[:, 0]
    onehot = (iota_f == idx_f).astype(jnp.float32)
    q = jax.lax.dot_general(
        onehot, cb, (((1,), (0,)), ((), ())),
        preferred_element_type=jnp.float32)
    q_ref[...] = q
    idx_ref[...] = idx[:, None]
    part = jnp.sum(dmin, axis=0, keepdims=True)
    prev = jnp.where(pl.program_id(0) == 0, 0.0, loss_ref[...])
    loss_ref[...] = prev + part


@functools.partial(jax.jit, static_argnames=())
def kernel(h_v_k, codebook):
    n, d = h_v_k.shape
    e = codebook.shape[0]
    cc = jnp.sum(codebook * codebook, axis=1)[None, :]
    grid = n // _BLOCK
    q, idx, loss = pl.pallas_call(
        _vq_block_kernel,
        grid=(grid,),
        in_specs=[
            pl.BlockSpec((_BLOCK, d), lambda i: (i, 0)),
            pl.BlockSpec((e, d), lambda i: (0, 0)),
            pl.BlockSpec((1, e), lambda i: (0, 0)),
        ],
        out_specs=[
            pl.BlockSpec((_BLOCK, d), lambda i: (i, 0)),
            pl.BlockSpec((_BLOCK, 1), lambda i: (i, 0)),
            pl.BlockSpec((1, 1), lambda i: (0, 0)),
        ],
        out_shape=[
            jax.ShapeDtypeStruct((n, d), jnp.float32),
            jax.ShapeDtypeStruct((n, 1), jnp.int32),
            jax.ShapeDtypeStruct((1, 1), jnp.float32),
        ],
    )(h_v_k, codebook, cc)
    total_loss = loss[0, 0] * ((1.0 + _COMMITMENT_COST) / (n * d))
    return (q, idx.reshape(n), total_loss)
